# Initial kernel scaffold; baseline (speedup 1.0000x reference)
import jax
import jax.numpy as jnp
from jax.experimental import pallas as pl


def kernel(edge_dist, W):
    raise NotImplementedError("write your pallas kernel here")


# speedup vs baseline: 18.5179x; 18.5179x over previous
import functools

import jax
import jax.numpy as jnp
from jax import lax
from jax.experimental import pallas as pl
from jax.experimental.pallas import tpu as pltpu
from jax.experimental.pallas import tpu_sc as plsc

B = 8
L = 512
H = 16
NUM_EMB = 514
SUPER = NUM_EMB - 1
TS = 520
ROWS_PER_ITEM = 4
ITEM = ROWS_PER_ITEM * L
N_ITEMS = (B * L) // ROWS_PER_ITEM
CHUNKS = ITEM // 16
CHUNKS_PER_ROW = L // 16


def _sc_lookup(edge_flat, wt_flat, n_workers):
    per_w = N_ITEMS // n_workers

    @functools.partial(
        pl.kernel,
        mesh=plsc.VectorSubcoreMesh(core_axis_name="c", subcore_axis_name="s"),
        compiler_params=pltpu.CompilerParams(needs_layout_passes=False),
        out_type=jax.ShapeDtypeStruct((B * H, L * L), jnp.float32),
        scratch_types=[
            pltpu.VMEM((H * TS,), jnp.float32),
            pltpu.VMEM((ITEM,), jnp.int32),
            pltpu.VMEM((ITEM,), jnp.int32),
            pltpu.VMEM((H, ITEM), jnp.float32),
            pltpu.VMEM((H, ITEM), jnp.float32),
            pltpu.SemaphoreType.DMA,
            pltpu.SemaphoreType.DMA,
            pltpu.SemaphoreType.DMA,
            pltpu.SemaphoreType.DMA,
        ],
    )
    def k(edge_hbm, wt_hbm, out_hbm, wt_v, ib0, ib1, ob0, ob1, si0, si1, so0, so1):
        nc = 2
        wid = lax.axis_index("s") * nc + lax.axis_index("c")
        base_item = wid * per_w
        lane0 = lax.iota(jnp.int32, 16) == 0

        def in_copy(it, buf, sem):
            item = base_item + it
            return pltpu.make_async_copy(
                edge_hbm.at[pl.ds(item * ITEM, ITEM)], buf, sem)

        def out_copy(it, buf, sem):
            item = base_item + it
            b = item // (L // ROWS_PER_ITEM)
            blk = item % (L // ROWS_PER_ITEM)
            return pltpu.make_async_copy(
                buf,
                out_hbm.at[pl.ds(b * H, H), pl.ds(blk * ITEM, ITEM)],
                sem)

        def compute(it, ibuf, obuf):
            item = base_item + it
            first_blk = (item % (L // ROWS_PER_ITEM)) == 0

            def body(c, carry):
                pos = ibuf[pl.ds(c * 16, 16)]
                j0 = (c & (CHUNKS_PER_ROW - 1)) == 0
                row0 = jnp.logical_and(first_blk, c < CHUNKS_PER_ROW)
                m = jnp.logical_or(jnp.logical_and(lane0, j0), row0)
                p0 = jnp.where(m, SUPER, pos)
                for h in range(H):
                    obuf[h, pl.ds(c * 16, 16)] = plsc.load_gather(
                        wt_v, [p0 + h * TS])
                return carry

            lax.fori_loop(0, CHUNKS, body, 0)

        pltpu.sync_copy(wt_hbm, wt_v)
        in_copy(0, ib0, si0).start()
        in_copy(1, ib1, si1).start()

        in_copy(0, ib0, si0).wait()
        compute(0, ib0, ob0)
        out_copy(0, ob0, so0).start()
        in_copy(2, ib0, si0).start()

        in_copy(1, ib1, si1).wait()
        compute(1, ib1, ob1)
        out_copy(1, ob1, so1).start()
        in_copy(3, ib1, si1).start()

        bufs = ((ib0, ob0, si0, so0), (ib1, ob1, si1, so1))

        def loop_body(t, carry):
            for p, (ibuf, obuf, si, so) in enumerate(bufs):
                it = 2 * t + p
                out_copy(it - 2, obuf, so).wait()
                in_copy(it, ibuf, si).wait()
                compute(it, ibuf, obuf)
                out_copy(it, obuf, so).start()
                in_copy(it + 2, ibuf, si).start()
            return carry

        lax.fori_loop(1, per_w // 2 - 1, loop_body, 0)

        for p, (ibuf, obuf, si, so) in enumerate(bufs):
            it = per_w - 2 + p
            out_copy(it - 2, obuf, so).wait()
            in_copy(it, ibuf, si).wait()
            compute(it, ibuf, obuf)
            out_copy(it, obuf, so).start()

        out_copy(per_w - 2, ob0, so0).wait()
        out_copy(per_w - 1, ob1, so1).wait()

    return k(edge_flat, wt_flat)


def kernel(edge_dist, W):
    info = plsc.get_sparse_core_info()
    n_workers = info.num_cores * info.num_subcores
    wt = jnp.zeros((H, TS), jnp.float32).at[:, :NUM_EMB].set(W.T)
    out2 = _sc_lookup(edge_dist.reshape(-1), wt.reshape(-1), n_workers)
    return out2.reshape(B, H, L, L)


# speedup vs baseline: 42.1367x; 2.2755x over previous
import functools

import jax
import jax.numpy as jnp
from jax import lax
from jax.experimental import pallas as pl
from jax.experimental.pallas import tpu as pltpu
from jax.experimental.pallas import tpu_sc as plsc

B = 8
L = 512
H = 16
NUM_EMB = 514
SUPER = NUM_EMB - 1
TS = 520
ROWS_PER_ITEM = 4
ITEM = ROWS_PER_ITEM * L
N_ITEMS = (B * L) // ROWS_PER_ITEM
CHUNKS = ITEM // 16
CHUNKS_PER_ROW = L // 16


def _sc_lookup(edge_flat, wt_flat, n_workers):
    per_w = N_ITEMS // n_workers

    @functools.partial(
        pl.kernel,
        mesh=plsc.VectorSubcoreMesh(core_axis_name="c", subcore_axis_name="s"),
        compiler_params=pltpu.CompilerParams(needs_layout_passes=False),
        out_type=jax.ShapeDtypeStruct((B * H, L * L), jnp.float32),
        scratch_types=[
            pltpu.VMEM((H * TS,), jnp.float32),
            pltpu.VMEM((ITEM,), jnp.int32),
            pltpu.VMEM((ITEM,), jnp.int32),
            pltpu.VMEM((H, ITEM), jnp.float32),
            pltpu.VMEM((H, ITEM), jnp.float32),
            pltpu.SemaphoreType.DMA,
            pltpu.SemaphoreType.DMA,
            pltpu.SemaphoreType.DMA,
            pltpu.SemaphoreType.DMA,
        ],
    )
    def k(edge_hbm, wt_hbm, out_hbm, wt_v, ib0, ib1, ob0, ob1, si0, si1, so0, so1):
        nc = 2
        wid = lax.axis_index("s") * nc + lax.axis_index("c")
        base_item = wid * per_w
        lane0 = lax.iota(jnp.int32, 16) == 0

        def in_copy(it, buf, sem):
            item = base_item + it
            return pltpu.make_async_copy(
                edge_hbm.at[pl.ds(item * ITEM, ITEM)], buf, sem)

        def out_copy(it, buf, sem):
            item = base_item + it
            b = item // (L // ROWS_PER_ITEM)
            blk = item % (L // ROWS_PER_ITEM)
            return pltpu.make_async_copy(
                buf,
                out_hbm.at[pl.ds(b * H, H), pl.ds(blk * ITEM, ITEM)],
                sem)

        super_vec = jnp.full((16,), SUPER, jnp.int32)

        def compute(it, ibuf, obuf):
            item = base_item + it
            first_blk = (item % (L // ROWS_PER_ITEM)) == 0

            for r in range(ROWS_PER_ITEM):
                v = ibuf[pl.ds(r * L, 16)]
                ibuf[pl.ds(r * L, 16)] = jnp.where(lane0, SUPER, v)

            @pl.when(first_blk)
            def _():
                @plsc.parallel_loop(0, CHUNKS_PER_ROW, unroll=4)
                def _(c):
                    ibuf[pl.ds(c * 16, 16)] = super_vec

            @plsc.parallel_loop(0, CHUNKS, unroll=4)
            def _(c):
                pos = ibuf[pl.ds(c * 16, 16)]
                for h in range(H):
                    obuf[h, pl.ds(c * 16, 16)] = plsc.load_gather(
                        wt_v, [pos + h * TS])

        pltpu.sync_copy(wt_hbm, wt_v)
        in_copy(0, ib0, si0).start()
        in_copy(1, ib1, si1).start()

        in_copy(0, ib0, si0).wait()
        compute(0, ib0, ob0)
        out_copy(0, ob0, so0).start()
        in_copy(2, ib0, si0).start()

        in_copy(1, ib1, si1).wait()
        compute(1, ib1, ob1)
        out_copy(1, ob1, so1).start()
        in_copy(3, ib1, si1).start()

        bufs = ((ib0, ob0, si0, so0), (ib1, ob1, si1, so1))

        def loop_body(t, carry):
            for p, (ibuf, obuf, si, so) in enumerate(bufs):
                it = 2 * t + p
                out_copy(it - 2, obuf, so).wait()
                in_copy(it, ibuf, si).wait()
                compute(it, ibuf, obuf)
                out_copy(it, obuf, so).start()
                in_copy(it + 2, ibuf, si).start()
            return carry

        lax.fori_loop(1, per_w // 2 - 1, loop_body, 0)

        for p, (ibuf, obuf, si, so) in enumerate(bufs):
            it = per_w - 2 + p
            out_copy(it - 2, obuf, so).wait()
            in_copy(it, ibuf, si).wait()
            compute(it, ibuf, obuf)
            out_copy(it, obuf, so).start()

        out_copy(per_w - 2, ob0, so0).wait()
        out_copy(per_w - 1, ob1, so1).wait()

    return k(edge_flat, wt_flat)


def kernel(edge_dist, W):
    info = plsc.get_sparse_core_info()
    n_workers = info.num_cores * info.num_subcores
    wt = jnp.zeros((H, TS), jnp.float32).at[:, :NUM_EMB].set(W.T)
    out2 = _sc_lookup(edge_dist.reshape(-1), wt.reshape(-1), n_workers)
    return out2.reshape(B, H, L, L)
